# baseline (speedup 1.0000x reference)
import functools

import jax
import jax.numpy as jnp
import numpy as np
from jax import lax
from jax.experimental import pallas as pl
from jax.experimental.pallas import tpu as pltpu
from jax.experimental.pallas import tpu_sc as plsc

_NC = 2
_NS = 16
_NW = _NC * _NS
_LANES = 16

_INT_MAX = np.int32(2147483647)
_IDX_MASK = np.int32(1023)
_KEY_MASK = np.int32(-1024)


def _knn_body(xt_ref, xs_ref, idx_ref, w_ref, *, s):
    b = pl.program_id(0)
    xt = xt_ref[0]
    xs = xs_ref[0]
    t2 = jnp.sum(xt * xt, axis=1, keepdims=True)
    s2 = jnp.sum(xs * xs, axis=0, keepdims=True)
    q = jnp.dot(xt, xs, preferred_element_type=jnp.float32)
    d = jnp.maximum(t2 + s2 - 2.0 * q, 0.0)
    row = lax.broadcasted_iota(jnp.int32, d.shape, 0)
    p = ((lax.bitcast_convert_type(d, jnp.int32) + np.int32(512))
         & _KEY_MASK) | row
    m1 = jnp.min(p, axis=0, keepdims=True)
    p = jnp.where(p == m1, _INT_MAX, p)
    m2 = jnp.min(p, axis=0, keepdims=True)
    p = jnp.where(p == m2, _INT_MAX, p)
    m3 = jnp.min(p, axis=0, keepdims=True)
    ms = (m1, m2, m3)
    rows = [m & _IDX_MASK for m in ms]
    dvals = [jnp.maximum(lax.bitcast_convert_type(m & _KEY_MASK, jnp.float32),
                         0.0) for m in ms]
    recips = [1.0 / (dv + 1e-8) for dv in dvals]
    norm = recips[0] + recips[1] + recips[2]
    ws = [r / norm for r in recips]
    base = b * s
    tn = d.shape[1]
    zi = jnp.zeros((5, tn), jnp.int32)
    zf = jnp.zeros((5, tn), jnp.float32)
    idx_ref[0] = jnp.concatenate([rows[0] + base, rows[1] + base,
                                  rows[2] + base, zi], axis=0)
    w_ref[0] = jnp.concatenate(ws + [zf], axis=0)


def _knn(xt_p, xs_t):
    bsz, s, _ = xt_p.shape
    n = xs_t.shape[2]
    tn = 512
    grid = (bsz, n // tn)
    idx, w = pl.pallas_call(
        functools.partial(_knn_body, s=s),
        grid=grid,
        in_specs=[
            pl.BlockSpec((1, s, 8), lambda b, i: (b, 0, 0)),
            pl.BlockSpec((1, 8, tn), lambda b, i: (b, 0, i)),
        ],
        out_specs=[
            pl.BlockSpec((1, 8, tn), lambda b, i: (b, 0, i)),
            pl.BlockSpec((1, 8, tn), lambda b, i: (b, 0, i)),
        ],
        out_shape=[
            jax.ShapeDtypeStruct((bsz, 8, n), jnp.int32),
            jax.ShapeDtypeStruct((bsz, 8, n), jnp.float32),
        ],
    )(xt_p, xs_t)
    return idx, w


def _interp_body(table_hbm, idx_hbm, w_hbm, out_hbm,
                 idx_v, w_v, rows_v, out_v, sem, *, n, n_chunks, cp, d):
    wid = lax.axis_index("s") * _NC + lax.axis_index("c")

    def chunk_body(c, carry):
        pbase = (wid * n_chunks + c) * cp
        b = pbase // n
        n0 = pbase % n
        pltpu.sync_copy(idx_hbm.at[b, :, pl.ds(n0, cp)], idx_v)
        pltpu.sync_copy(w_hbm.at[b, :, pl.ds(n0, cp)], w_v)
        copies = []
        for j in range(3):
            copies.append(
                pltpu.async_copy(table_hbm.at[idx_v.at[j]], rows_v.at[j], sem))
        for c_ in copies:
            c_.wait()

        def grp_body(g, carry2):
            p0 = g * _LANES
            w16 = [w_v[k, pl.ds(p0, _LANES)] for k in range(3)]
            for j in range(_LANES):
                p = p0 + j
                wvecs = [jnp.full((_LANES,), w16[k][j], jnp.float32)
                         for k in range(3)]
                for v in range(d // _LANES):
                    sl = pl.ds(v * _LANES, _LANES)
                    acc = wvecs[0] * rows_v[0, p, sl]
                    acc = acc + wvecs[1] * rows_v[1, p, sl]
                    acc = acc + wvecs[2] * rows_v[2, p, sl]
                    out_v[p, sl] = acc
            return carry2

        lax.fori_loop(0, cp // _LANES, grp_body, 0)
        pltpu.sync_copy(out_v, out_hbm.at[pl.ds(pbase, cp)])
        return carry

    lax.fori_loop(0, n_chunks, chunk_body, 0)


def _interp(table, idx, w, n_pts):
    d = table.shape[1]
    n = idx.shape[2]
    pts_w = n_pts // _NW
    cp = 128
    n_chunks = pts_w // cp
    mesh = plsc.VectorSubcoreMesh(core_axis_name="c", subcore_axis_name="s")
    kern = pl.kernel(
        functools.partial(_interp_body, n=n, n_chunks=n_chunks, cp=cp, d=d),
        out_type=jax.ShapeDtypeStruct((n_pts, d), jnp.float32),
        mesh=mesh,
        scratch_types=[
            pltpu.VMEM((8, cp), jnp.int32),
            pltpu.VMEM((8, cp), jnp.float32),
            pltpu.VMEM((3, cp, d), jnp.float32),
            pltpu.VMEM((cp, d), jnp.float32),
            pltpu.SemaphoreType.DMA,
        ],
    )
    return kern(table, idx, w)


def _mlp1_body(fs_ref, fi_ref, wa_ref, wb_ref, b_ref, y_ref, s_ref, q_ref):
    @pl.when(pl.program_id(0) == 0)
    def _():
        s_ref[...] = jnp.zeros_like(s_ref)
        q_ref[...] = jnp.zeros_like(q_ref)

    y = (jnp.dot(fs_ref[0], wa_ref[...], preferred_element_type=jnp.float32)
         + jnp.dot(fi_ref[...], wb_ref[...], preferred_element_type=jnp.float32)
         + b_ref[...])
    y_ref[...] = y
    s_ref[...] += jnp.sum(y, axis=0, keepdims=True)
    q_ref[...] += jnp.sum(y * y, axis=0, keepdims=True)


def _mlp2_body(y_ref, a_ref, c_ref, w_ref, b_ref, y2_ref, s_ref, q_ref):
    @pl.when(pl.program_id(0) == 0)
    def _():
        s_ref[...] = jnp.zeros_like(s_ref)
        q_ref[...] = jnp.zeros_like(q_ref)

    h = jnp.maximum(y_ref[...] * a_ref[...] + c_ref[...], 0.0)
    y2 = jnp.dot(h, w_ref[...], preferred_element_type=jnp.float32) + b_ref[...]
    y2_ref[...] = y2
    s_ref[...] += jnp.sum(y2, axis=0, keepdims=True)
    q_ref[...] += jnp.sum(y2 * y2, axis=0, keepdims=True)


def _affine_relu_body(y_ref, a_ref, c_ref, o_ref):
    o_ref[0] = jnp.maximum(y_ref[...] * a_ref[...] + c_ref[...], 0.0)


def _bn_affine(s, q, n, g, be):
    mean = s / n
    var = q / n - mean * mean
    a = g * lax.rsqrt(var + 1e-5)
    c = be - mean * a
    return a.reshape(1, -1), c.reshape(1, -1)


def _row_spec(tm, c):
    return pl.BlockSpec((tm, c), lambda i: (i, 0))


def _full_spec(shape):
    return pl.BlockSpec(shape, lambda i: tuple(0 for _ in shape))


def kernel(xyz_s, xyz_t, feats_s, feats_t, W1, b1, g1, be1, W2, b2, g2, be2):
    bsz, n, d = feats_s.shape
    s = feats_t.shape[1]
    n_pts = bsz * n

    xt_p = jnp.pad(xyz_t, ((0, 0), (0, 0), (0, 5)))
    xs_t = jnp.pad(jnp.transpose(xyz_s, (0, 2, 1)), ((0, 0), (0, 5), (0, 0)))

    idx, w = _knn(xt_p, xs_t)
    table = feats_t.reshape(bsz * s, d)
    inter = _interp(table, idx, w, n_pts)

    tm = 512
    npb = n // tm
    grid = (n_pts // tm,)
    c1 = W1.shape[0]
    c2 = W2.shape[0]

    y1, s1, q1 = pl.pallas_call(
        _mlp1_body,
        grid=grid,
        in_specs=[
            pl.BlockSpec((1, tm, d), lambda i: (i // npb, i % npb, 0)),
            _row_spec(tm, d),
            _full_spec((d, c1)), _full_spec((d, c1)), _full_spec((1, c1)),
        ],
        out_specs=[
            _row_spec(tm, c1), _full_spec((1, c1)), _full_spec((1, c1)),
        ],
        out_shape=[
            jax.ShapeDtypeStruct((n_pts, c1), jnp.float32),
            jax.ShapeDtypeStruct((1, c1), jnp.float32),
            jax.ShapeDtypeStruct((1, c1), jnp.float32),
        ],
    )(feats_s, inter, jnp.transpose(W1[:, :d]), jnp.transpose(W1[:, d:]),
      b1.reshape(1, c1))

    a1, cc1 = _bn_affine(s1[0], q1[0], float(n_pts), g1, be1)

    y2, s2, q2 = pl.pallas_call(
        _mlp2_body,
        grid=grid,
        in_specs=[
            _row_spec(tm, c1), _full_spec((1, c1)), _full_spec((1, c1)),
            _full_spec((c1, c2)), _full_spec((1, c2)),
        ],
        out_specs=[
            _row_spec(tm, c2), _full_spec((1, c2)), _full_spec((1, c2)),
        ],
        out_shape=[
            jax.ShapeDtypeStruct((n_pts, c2), jnp.float32),
            jax.ShapeDtypeStruct((1, c2), jnp.float32),
            jax.ShapeDtypeStruct((1, c2), jnp.float32),
        ],
    )(y1, a1, cc1, jnp.transpose(W2), b2.reshape(1, c2))

    a2, cc2 = _bn_affine(s2[0], q2[0], float(n_pts), g2, be2)

    out = pl.pallas_call(
        _affine_relu_body,
        grid=grid,
        in_specs=[_row_spec(tm, c2), _full_spec((1, c2)), _full_spec((1, c2))],
        out_specs=pl.BlockSpec((1, tm, c2), lambda i: (i // npb, i % npb, 0)),
        out_shape=jax.ShapeDtypeStruct((bsz, n, c2), jnp.float32),
    )(y2, a2, cc2)

    return out


# speedup vs baseline: 1.4838x; 1.4838x over previous
import functools

import jax
import jax.numpy as jnp
import numpy as np
from jax import lax
from jax.experimental import pallas as pl
from jax.experimental.pallas import tpu as pltpu
from jax.experimental.pallas import tpu_sc as plsc

_NC = 2
_NS = 16
_NW = _NC * _NS
_LANES = 16

_INT_MAX = np.int32(2147483647)
_IDX_MASK = np.int32(1023)
_KEY_MASK = np.int32(-1024)


def _knn_body(xt_ref, xs_ref, idx_ref, w_ref, *, s):
    b = pl.program_id(0)
    xt = xt_ref[0]
    xs = xs_ref[0]
    t2 = jnp.sum(xt * xt, axis=1, keepdims=True)
    s2 = jnp.sum(xs * xs, axis=0, keepdims=True)
    q = jnp.dot(xt, xs, preferred_element_type=jnp.float32)
    d = jnp.maximum(t2 + s2 - 2.0 * q, 0.0)
    row = lax.broadcasted_iota(jnp.int32, d.shape, 0)
    p = ((lax.bitcast_convert_type(d, jnp.int32) + np.int32(512))
         & _KEY_MASK) | row
    m1 = jnp.min(p, axis=0, keepdims=True)
    p = jnp.where(p == m1, _INT_MAX, p)
    m2 = jnp.min(p, axis=0, keepdims=True)
    p = jnp.where(p == m2, _INT_MAX, p)
    m3 = jnp.min(p, axis=0, keepdims=True)
    ms = (m1, m2, m3)
    rows = [m & _IDX_MASK for m in ms]
    dvals = [jnp.maximum(lax.bitcast_convert_type(m & _KEY_MASK, jnp.float32),
                         0.0) for m in ms]
    recips = [1.0 / (dv + 1e-8) for dv in dvals]
    norm = recips[0] + recips[1] + recips[2]
    ws = [r / norm for r in recips]
    base = b * s
    tn = d.shape[1]
    zi = jnp.zeros((5, tn), jnp.int32)
    zf = jnp.zeros((5, tn), jnp.float32)
    idx_ref[0] = jnp.concatenate([rows[0] + base, rows[1] + base,
                                  rows[2] + base, zi], axis=0)
    w_ref[0] = jnp.concatenate(ws + [zf], axis=0)


def _knn(xt_p, xs_t):
    bsz, s, _ = xt_p.shape
    n = xs_t.shape[2]
    tn = 1024
    grid = (bsz, n // tn)
    idx, w = pl.pallas_call(
        functools.partial(_knn_body, s=s),
        grid=grid,
        in_specs=[
            pl.BlockSpec((1, s, 8), lambda b, i: (b, 0, 0)),
            pl.BlockSpec((1, 8, tn), lambda b, i: (b, 0, i)),
        ],
        out_specs=[
            pl.BlockSpec((1, 8, tn), lambda b, i: (b, 0, i)),
            pl.BlockSpec((1, 8, tn), lambda b, i: (b, 0, i)),
        ],
        out_shape=[
            jax.ShapeDtypeStruct((bsz, 8, n), jnp.int32),
            jax.ShapeDtypeStruct((bsz, 8, n), jnp.float32),
        ],
    )(xt_p, xs_t)
    return idx, w


def _interp_body(table_hbm, idx_hbm, w_hbm, out_hbm,
                 idx_v, w_v, rows_v, out_v, sem0, sem1,
                 *, n, n_chunks, cp, d):
    wid = lax.axis_index("s") * _NC + lax.axis_index("c")
    base_chunk = wid * n_chunks
    sems = (sem0, sem1)

    def start(buf, c):
        pbase = (base_chunk + c) * cp
        b = pbase // n
        n0 = pbase % n
        pltpu.sync_copy(idx_hbm.at[b, :, pl.ds(n0, cp)], idx_v.at[buf])
        pltpu.sync_copy(w_hbm.at[b, :, pl.ds(n0, cp)], w_v.at[buf])
        for j in range(3):
            pltpu.async_copy(table_hbm.at[idx_v.at[buf, j]],
                             rows_v.at[buf, j], sems[buf])

    def finish(buf, c):
        pbase = (base_chunk + c) * cp
        for j in range(3):
            pltpu.make_async_copy(table_hbm.at[idx_v.at[buf, j]],
                                  rows_v.at[buf, j], sems[buf]).wait()

        def grp_body(g, carry2):
            p0 = g * _LANES
            w16 = [w_v[buf, k, pl.ds(p0, _LANES)] for k in range(3)]
            for j in range(_LANES):
                p = p0 + j
                wvecs = [jnp.full((_LANES,), w16[k][j], jnp.float32)
                         for k in range(3)]
                for v in range(d // _LANES):
                    sl = pl.ds(v * _LANES, _LANES)
                    acc = wvecs[0] * rows_v[buf, 0, p, sl]
                    acc = acc + wvecs[1] * rows_v[buf, 1, p, sl]
                    acc = acc + wvecs[2] * rows_v[buf, 2, p, sl]
                    out_v[p, sl] = acc
            return carry2

        lax.fori_loop(0, cp // _LANES, grp_body, 0)
        pltpu.sync_copy(out_v, out_hbm.at[pl.ds(pbase, cp)])

    n_pairs = n_chunks // 2
    start(0, 0)

    def pair_body(i, carry):
        c0 = 2 * i
        start(1, c0 + 1)
        finish(0, c0)

        @pl.when(i < n_pairs - 1)
        def _():
            start(0, c0 + 2)

        finish(1, c0 + 1)
        return carry

    lax.fori_loop(0, n_pairs, pair_body, 0)


def _interp(table, idx, w, n_pts):
    d = table.shape[1]
    n = idx.shape[2]
    pts_w = n_pts // _NW
    cp = 128
    n_chunks = pts_w // cp
    mesh = plsc.VectorSubcoreMesh(core_axis_name="c", subcore_axis_name="s")
    kern = pl.kernel(
        functools.partial(_interp_body, n=n, n_chunks=n_chunks, cp=cp, d=d),
        out_type=jax.ShapeDtypeStruct((n_pts, d), jnp.float32),
        mesh=mesh,
        scratch_types=[
            pltpu.VMEM((2, 8, cp), jnp.int32),
            pltpu.VMEM((2, 8, cp), jnp.float32),
            pltpu.VMEM((2, 3, cp, d), jnp.float32),
            pltpu.VMEM((cp, d), jnp.float32),
            pltpu.SemaphoreType.DMA,
            pltpu.SemaphoreType.DMA,
        ],
    )
    return kern(table, idx, w)


def _mlp1_body(fs_ref, fi_ref, wa_ref, b_ref, y_ref, s_ref, q_ref):
    @pl.when(pl.program_id(0) == 0)
    def _():
        s_ref[...] = jnp.zeros_like(s_ref)
        q_ref[...] = jnp.zeros_like(q_ref)

    x = jnp.concatenate([fs_ref[0], fi_ref[...]], axis=1)
    y = jnp.dot(x, wa_ref[...], preferred_element_type=jnp.float32) + b_ref[...]
    y_ref[...] = y
    s_ref[...] += jnp.sum(y, axis=0, keepdims=True)
    q_ref[...] += jnp.sum(y * y, axis=0, keepdims=True)


def _mlp2_body(y_ref, a_ref, c_ref, w_ref, b_ref, y2_ref, s_ref, q_ref):
    @pl.when(pl.program_id(0) == 0)
    def _():
        s_ref[...] = jnp.zeros_like(s_ref)
        q_ref[...] = jnp.zeros_like(q_ref)

    h = jnp.maximum(y_ref[...] * a_ref[...] + c_ref[...], 0.0)
    y2 = jnp.dot(h, w_ref[...], preferred_element_type=jnp.float32) + b_ref[...]
    y2_ref[...] = y2
    s_ref[...] += jnp.sum(y2, axis=0, keepdims=True)
    q_ref[...] += jnp.sum(y2 * y2, axis=0, keepdims=True)


def _affine_relu_body(y_ref, a_ref, c_ref, o_ref):
    o_ref[0] = jnp.maximum(y_ref[...] * a_ref[...] + c_ref[...], 0.0)


def _bn_affine(s, q, n, g, be):
    mean = s / n
    var = q / n - mean * mean
    a = g * lax.rsqrt(var + 1e-5)
    c = be - mean * a
    return a.reshape(1, -1), c.reshape(1, -1)


def _row_spec(tm, c):
    return pl.BlockSpec((tm, c), lambda i: (i, 0))


def _full_spec(shape):
    return pl.BlockSpec(shape, lambda i: tuple(0 for _ in shape))


def kernel(xyz_s, xyz_t, feats_s, feats_t, W1, b1, g1, be1, W2, b2, g2, be2):
    bsz, n, d = feats_s.shape
    s = feats_t.shape[1]
    n_pts = bsz * n

    xt_p = jnp.pad(xyz_t, ((0, 0), (0, 0), (0, 5)))
    xs_t = jnp.pad(jnp.transpose(xyz_s, (0, 2, 1)), ((0, 0), (0, 5), (0, 0)))

    idx, w = _knn(xt_p, xs_t)
    table = feats_t.reshape(bsz * s, d)
    inter = _interp(table, idx, w, n_pts)

    tm = 2048
    npb = n // tm
    grid = (n_pts // tm,)
    c1 = W1.shape[0]
    c2 = W2.shape[0]

    y1, s1, q1 = pl.pallas_call(
        _mlp1_body,
        grid=grid,
        in_specs=[
            pl.BlockSpec((1, tm, d), lambda i: (i // npb, i % npb, 0)),
            _row_spec(tm, d),
            _full_spec((2 * d, c1)), _full_spec((1, c1)),
        ],
        out_specs=[
            _row_spec(tm, c1), _full_spec((1, c1)), _full_spec((1, c1)),
        ],
        out_shape=[
            jax.ShapeDtypeStruct((n_pts, c1), jnp.float32),
            jax.ShapeDtypeStruct((1, c1), jnp.float32),
            jax.ShapeDtypeStruct((1, c1), jnp.float32),
        ],
    )(feats_s, inter, jnp.transpose(W1), b1.reshape(1, c1))

    a1, cc1 = _bn_affine(s1[0], q1[0], float(n_pts), g1, be1)

    y2, s2, q2 = pl.pallas_call(
        _mlp2_body,
        grid=grid,
        in_specs=[
            _row_spec(tm, c1), _full_spec((1, c1)), _full_spec((1, c1)),
            _full_spec((c1, c2)), _full_spec((1, c2)),
        ],
        out_specs=[
            _row_spec(tm, c2), _full_spec((1, c2)), _full_spec((1, c2)),
        ],
        out_shape=[
            jax.ShapeDtypeStruct((n_pts, c2), jnp.float32),
            jax.ShapeDtypeStruct((1, c2), jnp.float32),
            jax.ShapeDtypeStruct((1, c2), jnp.float32),
        ],
    )(y1, a1, cc1, jnp.transpose(W2), b2.reshape(1, c2))

    a2, cc2 = _bn_affine(s2[0], q2[0], float(n_pts), g2, be2)

    out = pl.pallas_call(
        _affine_relu_body,
        grid=grid,
        in_specs=[_row_spec(tm, c2), _full_spec((1, c2)), _full_spec((1, c2))],
        out_specs=pl.BlockSpec((1, tm, c2), lambda i: (i // npb, i % npb, 0)),
        out_shape=jax.ShapeDtypeStruct((bsz, n, c2), jnp.float32),
    )(y2, a2, cc2)

    return out


# speedup vs baseline: 1.5908x; 1.0721x over previous
import functools

import jax
import jax.numpy as jnp
import numpy as np
from jax import lax
from jax.experimental import pallas as pl
from jax.experimental.pallas import tpu as pltpu
from jax.experimental.pallas import tpu_sc as plsc

_NC = 2
_NS = 16
_NW = _NC * _NS
_LANES = 16

_INT_MAX = np.int32(2147483647)
_IDX_MASK = np.int32(1023)
_KEY_MASK = np.int32(-1024)


def _knn_body(xt_ref, xs_ref, idx_ref, w_ref, *, s):
    b = pl.program_id(0)
    xt = xt_ref[0]
    xs = xs_ref[0]
    t2 = jnp.sum(xt * xt, axis=1, keepdims=True)
    s2 = jnp.sum(xs * xs, axis=0, keepdims=True)
    q = jnp.dot(xt, xs, preferred_element_type=jnp.float32)
    d = t2 + s2 - 2.0 * q
    row = lax.broadcasted_iota(jnp.int32, d.shape, 0)
    p = lax.bitcast_convert_type(
        ((lax.bitcast_convert_type(d, jnp.int32) + np.int32(512))
         & _KEY_MASK) | row, jnp.float32)
    big = np.float32(3.0e38)
    m1 = jnp.min(p, axis=0, keepdims=True)
    p = jnp.where(p == m1, big, p)
    m2 = jnp.min(p, axis=0, keepdims=True)
    p = jnp.where(p == m2, big, p)
    m3 = jnp.min(p, axis=0, keepdims=True)
    ms = [lax.bitcast_convert_type(m, jnp.int32) for m in (m1, m2, m3)]
    rows = [m & _IDX_MASK for m in ms]
    dvals = [jnp.maximum(lax.bitcast_convert_type(m & _KEY_MASK, jnp.float32),
                         0.0) for m in ms]
    recips = [1.0 / (dv + 1e-8) for dv in dvals]
    norm = recips[0] + recips[1] + recips[2]
    ws = [r / norm for r in recips]
    base = b * s
    tn = d.shape[1]
    zi = jnp.zeros((5, tn), jnp.int32)
    zf = jnp.zeros((5, tn), jnp.float32)
    idx_ref[0] = jnp.concatenate([rows[0] + base, rows[1] + base,
                                  rows[2] + base, zi], axis=0)
    w_ref[0] = jnp.concatenate(ws + [zf], axis=0)


def _knn(xt_p, xs_t):
    bsz, s, _ = xt_p.shape
    n = xs_t.shape[2]
    tn = 1024
    grid = (bsz, n // tn)
    idx, w = pl.pallas_call(
        functools.partial(_knn_body, s=s),
        grid=grid,
        in_specs=[
            pl.BlockSpec((1, s, 8), lambda b, i: (b, 0, 0)),
            pl.BlockSpec((1, 8, tn), lambda b, i: (b, 0, i)),
        ],
        out_specs=[
            pl.BlockSpec((1, 8, tn), lambda b, i: (b, 0, i)),
            pl.BlockSpec((1, 8, tn), lambda b, i: (b, 0, i)),
        ],
        out_shape=[
            jax.ShapeDtypeStruct((bsz, 8, n), jnp.int32),
            jax.ShapeDtypeStruct((bsz, 8, n), jnp.float32),
        ],
    )(xt_p, xs_t)
    return idx, w


def _interp_body(table_hbm, idx_hbm, w_hbm, out_hbm,
                 idx_v, w_v, rows_v, out_v, sem0, sem1,
                 *, n, n_chunks, cp, d):
    wid = lax.axis_index("s") * _NC + lax.axis_index("c")
    base_chunk = wid * n_chunks
    sems = (sem0, sem1)

    def start(buf, c):
        pbase = (base_chunk + c) * cp
        b = pbase // n
        n0 = pbase % n
        pltpu.sync_copy(idx_hbm.at[b, :, pl.ds(n0, cp)], idx_v.at[buf])
        pltpu.sync_copy(w_hbm.at[b, :, pl.ds(n0, cp)], w_v.at[buf])
        for j in range(3):
            pltpu.async_copy(table_hbm.at[idx_v.at[buf, j]],
                             rows_v.at[buf, j], sems[buf])

    def finish(buf, c):
        pbase = (base_chunk + c) * cp
        for j in range(3):
            pltpu.make_async_copy(table_hbm.at[idx_v.at[buf, j]],
                                  rows_v.at[buf, j], sems[buf]).wait()

        def grp_body(g, carry2):
            p0 = g * _LANES
            w16 = [w_v[buf, k, pl.ds(p0, _LANES)] for k in range(3)]
            for j in range(_LANES):
                p = p0 + j
                wvecs = [jnp.full((_LANES,), w16[k][j], jnp.float32)
                         for k in range(3)]
                for v in range(d // _LANES):
                    sl = pl.ds(v * _LANES, _LANES)
                    acc = wvecs[0] * rows_v[buf, 0, p, sl]
                    acc = acc + wvecs[1] * rows_v[buf, 1, p, sl]
                    acc = acc + wvecs[2] * rows_v[buf, 2, p, sl]
                    out_v[p, sl] = acc
            return carry2

        lax.fori_loop(0, cp // _LANES, grp_body, 0)
        pltpu.sync_copy(out_v, out_hbm.at[pl.ds(pbase, cp)])

    n_pairs = n_chunks // 2
    start(0, 0)

    def pair_body(i, carry):
        c0 = 2 * i
        start(1, c0 + 1)
        finish(0, c0)

        @pl.when(i < n_pairs - 1)
        def _():
            start(0, c0 + 2)

        finish(1, c0 + 1)
        return carry

    lax.fori_loop(0, n_pairs, pair_body, 0)


def _interp(table, idx, w, n_pts):
    d = table.shape[1]
    n = idx.shape[2]
    pts_w = n_pts // _NW
    cp = 128
    n_chunks = pts_w // cp
    mesh = plsc.VectorSubcoreMesh(core_axis_name="c", subcore_axis_name="s")
    kern = pl.kernel(
        functools.partial(_interp_body, n=n, n_chunks=n_chunks, cp=cp, d=d),
        out_type=jax.ShapeDtypeStruct((n_pts, d), jnp.float32),
        mesh=mesh,
        scratch_types=[
            pltpu.VMEM((2, 8, cp), jnp.int32),
            pltpu.VMEM((2, 8, cp), jnp.float32),
            pltpu.VMEM((2, 3, cp, d), jnp.float32),
            pltpu.VMEM((cp, d), jnp.float32),
            pltpu.SemaphoreType.DMA,
            pltpu.SemaphoreType.DMA,
        ],
    )
    return kern(table, idx, w)


def _mlp1_body(fs_ref, fi_ref, wa_ref, b_ref, y_ref, s_ref, q_ref):
    @pl.when(pl.program_id(0) == 0)
    def _():
        s_ref[...] = jnp.zeros_like(s_ref)
        q_ref[...] = jnp.zeros_like(q_ref)

    x = jnp.concatenate([fs_ref[0], fi_ref[...]], axis=1)
    y = jnp.dot(x, wa_ref[...], preferred_element_type=jnp.float32) + b_ref[...]
    y_ref[...] = y
    s_ref[...] += jnp.sum(y, axis=0, keepdims=True)
    q_ref[...] += jnp.sum(y * y, axis=0, keepdims=True)


def _mlp2_body(y_ref, a_ref, c_ref, w_ref, b_ref, y2_ref, s_ref, q_ref):
    @pl.when(pl.program_id(0) == 0)
    def _():
        s_ref[...] = jnp.zeros_like(s_ref)
        q_ref[...] = jnp.zeros_like(q_ref)

    h = jnp.maximum(y_ref[...] * a_ref[...] + c_ref[...], 0.0)
    y2 = jnp.dot(h, w_ref[...], preferred_element_type=jnp.float32) + b_ref[...]
    y2_ref[...] = y2
    s_ref[...] += jnp.sum(y2, axis=0, keepdims=True)
    q_ref[...] += jnp.sum(y2 * y2, axis=0, keepdims=True)


def _affine_relu_body(y_ref, a_ref, c_ref, o_ref):
    o_ref[0] = jnp.maximum(y_ref[...] * a_ref[...] + c_ref[...], 0.0)


def _bn_affine(s, q, n, g, be):
    mean = s / n
    var = q / n - mean * mean
    a = g * lax.rsqrt(var + 1e-5)
    c = be - mean * a
    return a.reshape(1, -1), c.reshape(1, -1)


def _row_spec(tm, c):
    return pl.BlockSpec((tm, c), lambda i: (i, 0))


def _full_spec(shape):
    return pl.BlockSpec(shape, lambda i: tuple(0 for _ in shape))


def kernel(xyz_s, xyz_t, feats_s, feats_t, W1, b1, g1, be1, W2, b2, g2, be2):
    bsz, n, d = feats_s.shape
    s = feats_t.shape[1]
    n_pts = bsz * n

    xt_p = jnp.pad(xyz_t, ((0, 0), (0, 0), (0, 5)))
    xs_t = jnp.pad(jnp.transpose(xyz_s, (0, 2, 1)), ((0, 0), (0, 5), (0, 0)))

    idx, w = _knn(xt_p, xs_t)
    table = feats_t.reshape(bsz * s, d)
    inter = _interp(table, idx, w, n_pts)

    tm = 2048
    npb = n // tm
    grid = (n_pts // tm,)
    c1 = W1.shape[0]
    c2 = W2.shape[0]

    y1, s1, q1 = pl.pallas_call(
        _mlp1_body,
        grid=grid,
        in_specs=[
            pl.BlockSpec((1, tm, d), lambda i: (i // npb, i % npb, 0)),
            _row_spec(tm, d),
            _full_spec((2 * d, c1)), _full_spec((1, c1)),
        ],
        out_specs=[
            _row_spec(tm, c1), _full_spec((1, c1)), _full_spec((1, c1)),
        ],
        out_shape=[
            jax.ShapeDtypeStruct((n_pts, c1), jnp.float32),
            jax.ShapeDtypeStruct((1, c1), jnp.float32),
            jax.ShapeDtypeStruct((1, c1), jnp.float32),
        ],
    )(feats_s, inter, jnp.transpose(W1), b1.reshape(1, c1))

    a1, cc1 = _bn_affine(s1[0], q1[0], float(n_pts), g1, be1)

    y2, s2, q2 = pl.pallas_call(
        _mlp2_body,
        grid=grid,
        in_specs=[
            _row_spec(tm, c1), _full_spec((1, c1)), _full_spec((1, c1)),
            _full_spec((c1, c2)), _full_spec((1, c2)),
        ],
        out_specs=[
            _row_spec(tm, c2), _full_spec((1, c2)), _full_spec((1, c2)),
        ],
        out_shape=[
            jax.ShapeDtypeStruct((n_pts, c2), jnp.float32),
            jax.ShapeDtypeStruct((1, c2), jnp.float32),
            jax.ShapeDtypeStruct((1, c2), jnp.float32),
        ],
    )(y1, a1, cc1, jnp.transpose(W2), b2.reshape(1, c2))

    a2, cc2 = _bn_affine(s2[0], q2[0], float(n_pts), g2, be2)

    out = pl.pallas_call(
        _affine_relu_body,
        grid=grid,
        in_specs=[_row_spec(tm, c2), _full_spec((1, c2)), _full_spec((1, c2))],
        out_specs=pl.BlockSpec((1, tm, c2), lambda i: (i // npb, i % npb, 0)),
        out_shape=jax.ShapeDtypeStruct((bsz, n, c2), jnp.float32),
    )(y2, a2, cc2)

    return out


# speedup vs baseline: 1.7619x; 1.1076x over previous
import functools

import jax
import jax.numpy as jnp
import numpy as np
from jax import lax
from jax.experimental import pallas as pl
from jax.experimental.pallas import tpu as pltpu
from jax.experimental.pallas import tpu_sc as plsc

_NC = 2
_NS = 16
_NW = _NC * _NS
_LANES = 16

_INT_MAX = np.int32(2147483647)
_IDX_MASK = np.int32(1023)
_KEY_MASK = np.int32(-1024)


def _knn_body(xt_ref, xs_ref, idx_ref, *, s):
    b = pl.program_id(0)
    xt = xt_ref[0]
    xs = xs_ref[0]
    t2 = jnp.sum(xt * xt, axis=1, keepdims=True)
    s2 = jnp.sum(xs * xs, axis=0, keepdims=True)
    q = jnp.dot(xt, xs, preferred_element_type=jnp.float32)
    d = t2 + s2 - 2.0 * q
    row = lax.broadcasted_iota(jnp.int32, d.shape, 0)
    p = lax.bitcast_convert_type(
        ((lax.bitcast_convert_type(d, jnp.int32) + np.int32(512))
         & _KEY_MASK) | row, jnp.float32)
    big = np.float32(3.0e38)
    m1 = jnp.min(p, axis=0, keepdims=True)
    p = jnp.where(p == m1, big, p)
    m2 = jnp.min(p, axis=0, keepdims=True)
    p = jnp.where(p == m2, big, p)
    m3 = jnp.min(p, axis=0, keepdims=True)
    ms = [lax.bitcast_convert_type(m, jnp.int32) for m in (m1, m2, m3)]
    rows = [m & _IDX_MASK for m in ms]
    dvals = [jnp.maximum(lax.bitcast_convert_type(m & _KEY_MASK, jnp.float32),
                         0.0) for m in ms]
    recips = [1.0 / (dv + 1e-8) for dv in dvals]
    norm = recips[0] + recips[1] + recips[2]
    ws = [r / norm for r in recips]
    base = b * s
    tn = d.shape[1]
    zi = jnp.zeros((2, tn), jnp.int32)
    wbits = [lax.bitcast_convert_type(wv, jnp.int32) for wv in ws]
    idx_ref[0] = jnp.concatenate([rows[0] + base, rows[1] + base,
                                  rows[2] + base] + wbits + [zi], axis=0)


def _knn(xt_p, xs_t):
    bsz, s, _ = xt_p.shape
    n = xs_t.shape[2]
    tn = 1024
    grid = (bsz, n // tn)
    meta = pl.pallas_call(
        functools.partial(_knn_body, s=s),
        grid=grid,
        in_specs=[
            pl.BlockSpec((1, s, 8), lambda b, i: (b, 0, 0)),
            pl.BlockSpec((1, 8, tn), lambda b, i: (b, 0, i)),
        ],
        out_specs=pl.BlockSpec((1, 8, tn), lambda b, i: (b, 0, i)),
        out_shape=jax.ShapeDtypeStruct((bsz, 8, n), jnp.int32),
    )(xt_p, xs_t)
    return meta


def _interp_body(table_hbm, idx_hbm, out_hbm,
                 idx_v, rows_v, out_v, sem0, sem1,
                 *, n, n_chunks, cp, d):
    wid = lax.axis_index("s") * _NC + lax.axis_index("c")
    base_chunk = wid * n_chunks
    sems = (sem0, sem1)

    def start(buf, c):
        pbase = (base_chunk + c) * cp
        b = pbase // n
        n0 = pbase % n
        pltpu.sync_copy(idx_hbm.at[b, :, pl.ds(n0, cp)], idx_v.at[buf])
        for j in range(3):
            pltpu.async_copy(table_hbm.at[idx_v.at[buf, j]],
                             rows_v.at[buf, j], sems[buf])

    def finish(buf, c):
        pbase = (base_chunk + c) * cp
        for j in range(3):
            pltpu.make_async_copy(table_hbm.at[idx_v.at[buf, j]],
                                  rows_v.at[buf, j], sems[buf]).wait()

        def grp_body(g, carry2):
            p0 = g * _LANES
            w16 = [lax.bitcast_convert_type(
                idx_v[buf, 3 + k, pl.ds(p0, _LANES)], jnp.float32)
                for k in range(3)]
            for j in range(_LANES):
                p = p0 + j
                wvecs = [jnp.full((_LANES,), w16[k][j], jnp.float32)
                         for k in range(3)]
                for v in range(d // _LANES):
                    sl = pl.ds(v * _LANES, _LANES)
                    acc = wvecs[0] * rows_v[buf, 0, p, sl]
                    acc = acc + wvecs[1] * rows_v[buf, 1, p, sl]
                    acc = acc + wvecs[2] * rows_v[buf, 2, p, sl]
                    out_v[p, sl] = acc
            return carry2

        lax.fori_loop(0, cp // _LANES, grp_body, 0)
        pltpu.sync_copy(out_v, out_hbm.at[pl.ds(pbase, cp)])

    n_pairs = n_chunks // 2
    start(0, 0)

    def pair_body(i, carry):
        c0 = 2 * i
        start(1, c0 + 1)
        finish(0, c0)

        @pl.when(i < n_pairs - 1)
        def _():
            start(0, c0 + 2)

        finish(1, c0 + 1)
        return carry

    lax.fori_loop(0, n_pairs, pair_body, 0)


def _interp(table, idx, n_pts):
    d = table.shape[1]
    n = idx.shape[2]
    pts_w = n_pts // _NW
    cp = 128
    n_chunks = pts_w // cp
    mesh = plsc.VectorSubcoreMesh(core_axis_name="c", subcore_axis_name="s")
    kern = pl.kernel(
        functools.partial(_interp_body, n=n, n_chunks=n_chunks, cp=cp, d=d),
        out_type=jax.ShapeDtypeStruct((n_pts, d), jnp.float32),
        mesh=mesh,
        scratch_types=[
            pltpu.VMEM((2, 8, cp), jnp.int32),
            pltpu.VMEM((2, 3, cp, d), jnp.float32),
            pltpu.VMEM((cp, d), jnp.float32),
            pltpu.SemaphoreType.DMA,
            pltpu.SemaphoreType.DMA,
        ],
    )
    return kern(table, idx)


def _mlp1_body(fs_ref, fi_ref, wa_ref, b_ref, y_ref, s_ref, q_ref):
    @pl.when(pl.program_id(0) == 0)
    def _():
        s_ref[...] = jnp.zeros_like(s_ref)
        q_ref[...] = jnp.zeros_like(q_ref)

    x = jnp.concatenate([fs_ref[0], fi_ref[...]], axis=1)
    y = jnp.dot(x, wa_ref[...], preferred_element_type=jnp.float32) + b_ref[...]
    y_ref[...] = y
    s_ref[...] += jnp.sum(y, axis=0, keepdims=True)
    q_ref[...] += jnp.sum(y * y, axis=0, keepdims=True)


def _mlp2_body(y_ref, a_ref, c_ref, w_ref, b_ref, y2_ref, s_ref, q_ref):
    @pl.when(pl.program_id(0) == 0)
    def _():
        s_ref[...] = jnp.zeros_like(s_ref)
        q_ref[...] = jnp.zeros_like(q_ref)

    h = jnp.maximum(y_ref[...] * a_ref[...] + c_ref[...], 0.0)
    y2 = jnp.dot(h, w_ref[...], preferred_element_type=jnp.float32) + b_ref[...]
    y2_ref[...] = y2
    s_ref[...] += jnp.sum(y2, axis=0, keepdims=True)
    q_ref[...] += jnp.sum(y2 * y2, axis=0, keepdims=True)


def _affine_relu_body(y_ref, a_ref, c_ref, o_ref):
    o_ref[0] = jnp.maximum(y_ref[...] * a_ref[...] + c_ref[...], 0.0)


def _bn_affine(s, q, n, g, be):
    mean = s / n
    var = q / n - mean * mean
    a = g * lax.rsqrt(var + 1e-5)
    c = be - mean * a
    return a.reshape(1, -1), c.reshape(1, -1)


def _row_spec(tm, c):
    return pl.BlockSpec((tm, c), lambda i: (i, 0))


def _full_spec(shape):
    return pl.BlockSpec(shape, lambda i: tuple(0 for _ in shape))


def kernel(xyz_s, xyz_t, feats_s, feats_t, W1, b1, g1, be1, W2, b2, g2, be2):
    bsz, n, d = feats_s.shape
    s = feats_t.shape[1]
    n_pts = bsz * n

    xt_p = jnp.pad(xyz_t, ((0, 0), (0, 0), (0, 5)))
    xs_t = jnp.pad(jnp.transpose(xyz_s, (0, 2, 1)), ((0, 0), (0, 5), (0, 0)))

    meta = _knn(xt_p, xs_t)
    table = feats_t.reshape(bsz * s, d)
    inter = _interp(table, meta, n_pts)

    tm = 4096
    npb = n // tm
    grid = (n_pts // tm,)
    c1 = W1.shape[0]
    c2 = W2.shape[0]

    y1, s1, q1 = pl.pallas_call(
        _mlp1_body,
        grid=grid,
        in_specs=[
            pl.BlockSpec((1, tm, d), lambda i: (i // npb, i % npb, 0)),
            _row_spec(tm, d),
            _full_spec((2 * d, c1)), _full_spec((1, c1)),
        ],
        out_specs=[
            _row_spec(tm, c1), _full_spec((1, c1)), _full_spec((1, c1)),
        ],
        out_shape=[
            jax.ShapeDtypeStruct((n_pts, c1), jnp.float32),
            jax.ShapeDtypeStruct((1, c1), jnp.float32),
            jax.ShapeDtypeStruct((1, c1), jnp.float32),
        ],
    )(feats_s, inter, jnp.transpose(W1), b1.reshape(1, c1))

    a1, cc1 = _bn_affine(s1[0], q1[0], float(n_pts), g1, be1)

    y2, s2, q2 = pl.pallas_call(
        _mlp2_body,
        grid=grid,
        in_specs=[
            _row_spec(tm, c1), _full_spec((1, c1)), _full_spec((1, c1)),
            _full_spec((c1, c2)), _full_spec((1, c2)),
        ],
        out_specs=[
            _row_spec(tm, c2), _full_spec((1, c2)), _full_spec((1, c2)),
        ],
        out_shape=[
            jax.ShapeDtypeStruct((n_pts, c2), jnp.float32),
            jax.ShapeDtypeStruct((1, c2), jnp.float32),
            jax.ShapeDtypeStruct((1, c2), jnp.float32),
        ],
    )(y1, a1, cc1, jnp.transpose(W2), b2.reshape(1, c2))

    a2, cc2 = _bn_affine(s2[0], q2[0], float(n_pts), g2, be2)

    out = pl.pallas_call(
        _affine_relu_body,
        grid=grid,
        in_specs=[_row_spec(tm, c2), _full_spec((1, c2)), _full_spec((1, c2))],
        out_specs=pl.BlockSpec((1, tm, c2), lambda i: (i // npb, i % npb, 0)),
        out_shape=jax.ShapeDtypeStruct((bsz, n, c2), jnp.float32),
    )(y2, a2, cc2)

    return out
